# baseline (speedup 1.0000x reference)
import jax
import jax.numpy as jnp
from jax import lax
from jax.experimental import pallas as pl
from jax.experimental.pallas import tpu as pltpu
from jax.experimental.pallas import tpu_sc as plsc

N = 10000
E = 320000
D = 128

NC = 2
NS = 16
NW = NC * NS
EPW = E // NW
CHUNK = 80
NPH = 2
CPP = 64
NCHUNK = NPH * CPP
PAD = NCHUNK * CHUNK - EPW
ACC_N = 10112
RPT = ACC_N // NS


def _sc_agg_body(src_hbm, dst_hbm, x_hbm, zeros_hbm, out_hbm,
                 src_v, dst_v, rows_a, rows_b, acc_sh, sem_a, sem_b):
    c = lax.axis_index("c")
    s = lax.axis_index("s")
    wid = c * NS + s

    pltpu.sync_copy(zeros_hbm.at[s], acc_sh.at[pl.ds(s * RPT, RPT)])
    plsc.subcore_barrier()

    def gather(j, rows, sem):
        return pltpu.async_copy(x_hbm.at[src_v.at[j]], rows, sem)

    def scatter(j, rows):
        pltpu.sync_copy(rows, acc_sh.at[dst_v.at[j]], add=True)

    for p in range(NPH):
        pltpu.sync_copy(src_hbm.at[wid, p], src_v)
        pltpu.sync_copy(dst_hbm.at[wid, p], dst_v)

        gather(0, rows_a, sem_a).wait()

        def body(jj, carry):
            j = 2 * jj
            d_b = gather(j + 1, rows_b, sem_b)
            scatter(j, rows_a)
            d_b.wait()
            d_a = gather(j + 2, rows_a, sem_a)
            scatter(j + 1, rows_b)
            d_a.wait()
            return carry

        lax.fori_loop(0, CPP // 2 - 1, body, 0)
        d_b = gather(CPP - 1, rows_b, sem_b)
        scatter(CPP - 2, rows_a)
        d_b.wait()
        scatter(CPP - 1, rows_b)

    plsc.subcore_barrier()
    pltpu.sync_copy(acc_sh.at[pl.ds(s * RPT, RPT)], out_hbm.at[c, s])


@jax.jit
def _sc_agg(src4d, dst4d, x, zeros):
    mesh = plsc.VectorSubcoreMesh(core_axis_name="c", subcore_axis_name="s",
                                  num_cores=NC, num_subcores=NS)
    f = pl.kernel(
        _sc_agg_body,
        out_type=jax.ShapeDtypeStruct((NC, NS, RPT, D), jnp.float32),
        mesh=mesh,
        scratch_types=[
            pltpu.VMEM((CPP, CHUNK), jnp.int32),
            pltpu.VMEM((CPP, CHUNK), jnp.int32),
            pltpu.VMEM((CHUNK, D), jnp.float32),
            pltpu.VMEM((CHUNK, D), jnp.float32),
            pltpu.VMEM_SHARED((ACC_N, D), jnp.float32),
            pltpu.SemaphoreType.DMA,
            pltpu.SemaphoreType.DMA,
        ],
    )
    return f(src4d, dst4d, x, zeros)


def _tc_mlp_body(x_ref, p_ref, W1_ref, b1_ref, W2_ref, b2_ref, W3_ref, b3_ref,
                 gamma_ref, beta_ref, o_ref):
    h = x_ref[...] + p_ref[0, :N] + p_ref[1, :N]
    h = jax.nn.sigmoid(
        jnp.dot(h, W1_ref[...], preferred_element_type=jnp.float32)
        + b1_ref[...])
    h = jax.nn.sigmoid(
        jnp.dot(h, W2_ref[...], preferred_element_type=jnp.float32)
        + b2_ref[...])
    h = (jnp.dot(h, W3_ref[...], preferred_element_type=jnp.float32)
         + b3_ref[...])
    h = jnp.where(h >= 0, h, 0.01 * h)
    mean = jnp.mean(h, axis=0, keepdims=True)
    var = jnp.mean(h * h, axis=0, keepdims=True) - mean * mean
    o_ref[...] = ((h - mean) * jax.lax.rsqrt(var + 1e-5) * gamma_ref[...]
                  + beta_ref[...])


@jax.jit
def _tc_mlp(x, partials, W1, b1, W2, b2, W3, b3, gamma, beta):
    return pl.pallas_call(
        _tc_mlp_body,
        out_shape=jax.ShapeDtypeStruct((N, D), jnp.float32),
    )(x, partials, W1, b1.reshape(1, -1), W2, b2.reshape(1, -1),
      W3, b3.reshape(1, -1), gamma.reshape(1, -1), beta.reshape(1, -1))


@jax.jit
def kernel(x, edge_index, W1, b1, W2, b2, W3, b3, gamma, beta):
    src = edge_index[0].reshape(NW, EPW)
    dst = edge_index[1].reshape(NW, EPW)
    junk = N + (4 * jnp.arange(NW, dtype=jnp.int32)[:, None]
                + jnp.arange(PAD, dtype=jnp.int32)[None, :]) % (ACC_N - N)
    src4d = jnp.pad(src, ((0, 0), (0, PAD))).reshape(NW, NPH, CPP, CHUNK)
    dst4d = jnp.concatenate([dst, junk], axis=1).reshape(NW, NPH, CPP, CHUNK)
    zeros = jnp.zeros((NS, RPT, D), jnp.float32)
    out4d = _sc_agg(src4d, dst4d, x, zeros)
    partials = out4d.reshape(NC, ACC_N, D)
    h = _tc_mlp(x, partials, W1, b1, W2, b2, W3, b3, gamma, beta)
    return (h, edge_index)


# speedup vs baseline: 1.8624x; 1.8624x over previous
import jax
import jax.numpy as jnp
from jax import lax
from jax.experimental import pallas as pl
from jax.experimental.pallas import tpu as pltpu
from jax.experimental.pallas import tpu_sc as plsc

N = 10000
E = 320000
D = 128

NC = 2
NS = 16
NW = NC * NS
EPW = E // NW
CHUNK = 80
NCHUNK = EPW // CHUNK
ACC_N = 10240
RPT = ACC_N // NS


def _sc_agg_body(src_hbm, dst_hbm, x_hbm, zeros_hbm, out_hbm,
                 src_v, dst_v, rows_v, acc_sh, sem):
    c = lax.axis_index("c")
    s = lax.axis_index("s")
    wid = c * NS + s

    pltpu.sync_copy(zeros_hbm.at[s], acc_sh.at[pl.ds(s * RPT, RPT)])
    pltpu.sync_copy(src_hbm.at[wid], src_v)
    pltpu.sync_copy(dst_hbm.at[wid], dst_v)
    plsc.subcore_barrier()

    def body(j, carry):
        pltpu.async_copy(x_hbm.at[src_v.at[j]], rows_v, sem).wait()
        pltpu.sync_copy(rows_v, acc_sh.at[dst_v.at[j]], add=True)
        return carry

    lax.fori_loop(0, NCHUNK, body, 0)
    plsc.subcore_barrier()

    pltpu.sync_copy(acc_sh.at[pl.ds(s * RPT, RPT)], out_hbm.at[c, s])


@jax.jit
def _sc_agg(src3d, dst3d, x, zeros):
    mesh = plsc.VectorSubcoreMesh(core_axis_name="c", subcore_axis_name="s",
                                  num_cores=NC, num_subcores=NS)
    f = pl.kernel(
        _sc_agg_body,
        out_type=jax.ShapeDtypeStruct((NC, NS, RPT, D), jnp.float32),
        mesh=mesh,
        scratch_types=[
            pltpu.VMEM((NCHUNK, CHUNK), jnp.int32),
            pltpu.VMEM((NCHUNK, CHUNK), jnp.int32),
            pltpu.VMEM((CHUNK, D), jnp.float32),
            pltpu.VMEM_SHARED((ACC_N, D), jnp.float32),
            pltpu.SemaphoreType.DMA,
        ],
    )
    return f(src3d, dst3d, x, zeros)


def _tc_mlp_body(x_ref, p_ref, W1_ref, b1_ref, W2_ref, b2_ref, W3_ref, b3_ref,
                 gamma_ref, beta_ref, o_ref):
    h = x_ref[...] + p_ref[0] + p_ref[1]
    h = jax.nn.sigmoid(
        jnp.dot(h, W1_ref[...], preferred_element_type=jnp.float32)
        + b1_ref[...])
    h = jax.nn.sigmoid(
        jnp.dot(h, W2_ref[...], preferred_element_type=jnp.float32)
        + b2_ref[...])
    h = (jnp.dot(h, W3_ref[...], preferred_element_type=jnp.float32)
         + b3_ref[...])
    h = jnp.where(h >= 0, h, 0.01 * h)
    mean = jnp.mean(h, axis=0, keepdims=True)
    var = jnp.mean(h * h, axis=0, keepdims=True) - mean * mean
    o_ref[...] = ((h - mean) * jax.lax.rsqrt(var + 1e-5) * gamma_ref[...]
                  + beta_ref[...])


@jax.jit
def _tc_mlp(x, partials, W1, b1, W2, b2, W3, b3, gamma, beta):
    return pl.pallas_call(
        _tc_mlp_body,
        out_shape=jax.ShapeDtypeStruct((N, D), jnp.float32),
    )(x, partials, W1, b1.reshape(1, -1), W2, b2.reshape(1, -1),
      W3, b3.reshape(1, -1), gamma.reshape(1, -1), beta.reshape(1, -1))


def kernel(x, edge_index, W1, b1, W2, b2, W3, b3, gamma, beta):
    src3d = edge_index[0].reshape(NW, NCHUNK, CHUNK)
    dst3d = edge_index[1].reshape(NW, NCHUNK, CHUNK)
    zeros = jnp.zeros((NS, RPT, D), jnp.float32)
    out4d = _sc_agg(src3d, dst3d, x, zeros)
    partials = out4d.reshape(NC, ACC_N, D)[:, :N]
    h = _tc_mlp(x, partials, W1, b1, W2, b2, W3, b3, gamma, beta)
    return (h, edge_index)


# speedup vs baseline: 2.2285x; 1.1966x over previous
import jax
import jax.numpy as jnp
from jax import lax
from jax.experimental import pallas as pl
from jax.experimental.pallas import tpu as pltpu
from jax.experimental.pallas import tpu_sc as plsc

N = 10000
E = 320000
D = 128

NC = 2
NS = 16
NW = NC * NS
EPW = E // NW
CHUNK = 80
NCHUNK = EPW // CHUNK
NPH = 5
CPP = NCHUNK // NPH
ACC_N = 10240
RPT = ACC_N // NS


def _sc_agg_body(src_hbm, dst_hbm, x_hbm, zeros_hbm, out_hbm,
                 src_v, dst_v, rows_a, rows_b, acc_sh, sem_a, sem_b):
    c = lax.axis_index("c")
    s = lax.axis_index("s")
    wid = c * NS + s

    pltpu.sync_copy(zeros_hbm.at[s], acc_sh.at[pl.ds(s * RPT, RPT)])
    plsc.subcore_barrier()

    def gather(j, rows, sem):
        return pltpu.async_copy(x_hbm.at[src_v.at[j]], rows, sem)

    def scatter(j, rows):
        pltpu.sync_copy(rows, acc_sh.at[dst_v.at[j]], add=True)

    for p in range(NPH):
        pltpu.sync_copy(src_hbm.at[wid, p], src_v)
        pltpu.sync_copy(dst_hbm.at[wid, p], dst_v)

        gather(0, rows_a, sem_a).wait()

        def body(jj, carry):
            j = 2 * jj
            d_b = gather(j + 1, rows_b, sem_b)
            scatter(j, rows_a)
            d_b.wait()
            d_a = gather(j + 2, rows_a, sem_a)
            scatter(j + 1, rows_b)
            d_a.wait()
            return carry

        lax.fori_loop(0, (CPP - 1) // 2, body, 0)
        scatter(CPP - 1, rows_a)

    plsc.subcore_barrier()

    pltpu.sync_copy(acc_sh.at[pl.ds(s * RPT, RPT)], out_hbm.at[c, s])


@jax.jit
def _sc_agg(src3d, dst3d, x, zeros):
    mesh = plsc.VectorSubcoreMesh(core_axis_name="c", subcore_axis_name="s",
                                  num_cores=NC, num_subcores=NS)
    f = pl.kernel(
        _sc_agg_body,
        out_type=jax.ShapeDtypeStruct((NC, NS, RPT, D), jnp.float32),
        mesh=mesh,
        scratch_types=[
            pltpu.VMEM((CPP, CHUNK), jnp.int32),
            pltpu.VMEM((CPP, CHUNK), jnp.int32),
            pltpu.VMEM((CHUNK, D), jnp.float32),
            pltpu.VMEM((CHUNK, D), jnp.float32),
            pltpu.VMEM_SHARED((ACC_N, D), jnp.float32),
            pltpu.SemaphoreType.DMA,
            pltpu.SemaphoreType.DMA,
        ],
    )
    return f(src3d, dst3d, x, zeros)


def _tc_mlp_body(x_ref, p_ref, W1_ref, b1_ref, W2_ref, b2_ref, W3_ref, b3_ref,
                 gamma_ref, beta_ref, o_ref):
    h = x_ref[...] + p_ref[0] + p_ref[1]
    h = jax.nn.sigmoid(
        jnp.dot(h, W1_ref[...], preferred_element_type=jnp.float32)
        + b1_ref[...])
    h = jax.nn.sigmoid(
        jnp.dot(h, W2_ref[...], preferred_element_type=jnp.float32)
        + b2_ref[...])
    h = (jnp.dot(h, W3_ref[...], preferred_element_type=jnp.float32)
         + b3_ref[...])
    h = jnp.where(h >= 0, h, 0.01 * h)
    mean = jnp.mean(h, axis=0, keepdims=True)
    var = jnp.mean(h * h, axis=0, keepdims=True) - mean * mean
    o_ref[...] = ((h - mean) * jax.lax.rsqrt(var + 1e-5) * gamma_ref[...]
                  + beta_ref[...])


@jax.jit
def _tc_mlp(x, partials, W1, b1, W2, b2, W3, b3, gamma, beta):
    return pl.pallas_call(
        _tc_mlp_body,
        out_shape=jax.ShapeDtypeStruct((N, D), jnp.float32),
    )(x, partials, W1, b1.reshape(1, -1), W2, b2.reshape(1, -1),
      W3, b3.reshape(1, -1), gamma.reshape(1, -1), beta.reshape(1, -1))


def kernel(x, edge_index, W1, b1, W2, b2, W3, b3, gamma, beta):
    src3d = edge_index[0].reshape(NW, NPH, CPP, CHUNK)
    dst3d = edge_index[1].reshape(NW, NPH, CPP, CHUNK)
    zeros = jnp.zeros((NS, RPT, D), jnp.float32)
    out4d = _sc_agg(src3d, dst3d, x, zeros)
    partials = out4d.reshape(NC, ACC_N, D)[:, :N]
    h = _tc_mlp(x, partials, W1, b1, W2, b2, W3, b3, gamma, beta)
    return (h, edge_index)


# speedup vs baseline: 2.3147x; 1.0387x over previous
import jax
import jax.numpy as jnp
from jax import lax
from jax.experimental import pallas as pl
from jax.experimental.pallas import tpu as pltpu
from jax.experimental.pallas import tpu_sc as plsc

N = 10000
E = 320000
D = 128

NC = 2
NS = 16
NW = NC * NS
EPW = E // NW
CHUNK = 80
NCHUNK = EPW // CHUNK
NPH = 5
CPP = NCHUNK // NPH
ACC_N = 10240
RPT = ACC_N // NS


def _sc_agg_body(src_hbm, dst_hbm, x_hbm, zeros_hbm, out_hbm,
                 src_v, dst_v, rows_a, rows_b, acc_sh, sem_a, sem_b):
    c = lax.axis_index("c")
    s = lax.axis_index("s")
    wid = c * NS + s

    pltpu.sync_copy(zeros_hbm.at[s], acc_sh.at[pl.ds(s * RPT, RPT)])
    plsc.subcore_barrier()

    def gather(j, rows, sem):
        return pltpu.async_copy(x_hbm.at[src_v.at[j]], rows, sem)

    def scatter(j, rows):
        pltpu.sync_copy(rows, acc_sh.at[dst_v.at[j]], add=True)

    for p in range(NPH):
        pltpu.sync_copy(src_hbm.at[wid, p], src_v)
        pltpu.sync_copy(dst_hbm.at[wid, p], dst_v)

        gather(0, rows_a, sem_a).wait()

        def body(jj, carry):
            j = 2 * jj
            d_b = gather(j + 1, rows_b, sem_b)
            scatter(j, rows_a)
            d_b.wait()
            d_a = gather(j + 2, rows_a, sem_a)
            scatter(j + 1, rows_b)
            d_a.wait()
            return carry

        lax.fori_loop(0, (CPP - 1) // 2, body, 0)
        scatter(CPP - 1, rows_a)

    plsc.subcore_barrier()

    pltpu.sync_copy(acc_sh.at[pl.ds(s * RPT, RPT)], out_hbm.at[c, s])


@jax.jit
def _sc_agg(src3d, dst3d, x, zeros):
    mesh = plsc.VectorSubcoreMesh(core_axis_name="c", subcore_axis_name="s",
                                  num_cores=NC, num_subcores=NS)
    f = pl.kernel(
        _sc_agg_body,
        out_type=jax.ShapeDtypeStruct((NC, NS, RPT, D), jnp.float32),
        mesh=mesh,
        scratch_types=[
            pltpu.VMEM((CPP, CHUNK), jnp.int32),
            pltpu.VMEM((CPP, CHUNK), jnp.int32),
            pltpu.VMEM((CHUNK, D), jnp.float32),
            pltpu.VMEM((CHUNK, D), jnp.float32),
            pltpu.VMEM_SHARED((ACC_N, D), jnp.float32),
            pltpu.SemaphoreType.DMA,
            pltpu.SemaphoreType.DMA,
        ],
    )
    return f(src3d, dst3d, x, zeros)


def _tc_mlp_body(x_ref, p_ref, W1_ref, b1_ref, W2_ref, b2_ref, W3_ref, b3_ref,
                 gamma_ref, beta_ref, o_ref):
    h = x_ref[...] + p_ref[0, :N] + p_ref[1, :N]
    h = jax.nn.sigmoid(
        jnp.dot(h, W1_ref[...], preferred_element_type=jnp.float32)
        + b1_ref[...])
    h = jax.nn.sigmoid(
        jnp.dot(h, W2_ref[...], preferred_element_type=jnp.float32)
        + b2_ref[...])
    h = (jnp.dot(h, W3_ref[...], preferred_element_type=jnp.float32)
         + b3_ref[...])
    h = jnp.where(h >= 0, h, 0.01 * h)
    mean = jnp.mean(h, axis=0, keepdims=True)
    var = jnp.mean(h * h, axis=0, keepdims=True) - mean * mean
    o_ref[...] = ((h - mean) * jax.lax.rsqrt(var + 1e-5) * gamma_ref[...]
                  + beta_ref[...])


@jax.jit
def _tc_mlp(x, partials, W1, b1, W2, b2, W3, b3, gamma, beta):
    return pl.pallas_call(
        _tc_mlp_body,
        out_shape=jax.ShapeDtypeStruct((N, D), jnp.float32),
    )(x, partials, W1, b1.reshape(1, -1), W2, b2.reshape(1, -1),
      W3, b3.reshape(1, -1), gamma.reshape(1, -1), beta.reshape(1, -1))


@jax.jit
def kernel(x, edge_index, W1, b1, W2, b2, W3, b3, gamma, beta):
    src3d = edge_index[0].reshape(NW, NPH, CPP, CHUNK)
    dst3d = edge_index[1].reshape(NW, NPH, CPP, CHUNK)
    zeros = jnp.zeros((NS, RPT, D), jnp.float32)
    out4d = _sc_agg(src3d, dst3d, x, zeros)
    partials = out4d.reshape(NC, ACC_N, D)
    h = _tc_mlp(x, partials, W1, b1, W2, b2, W3, b3, gamma, beta)
    return (h, edge_index)


# speedup vs baseline: 2.5714x; 1.1109x over previous
import jax
import jax.numpy as jnp
from jax import lax
from jax.experimental import pallas as pl
from jax.experimental.pallas import tpu as pltpu
from jax.experimental.pallas import tpu_sc as plsc

N = 10000
E = 320000
D = 128

NC = 2
NS = 16
NW = NC * NS
EPW = E // NW
CHUNK = 80
NCHUNK = EPW // CHUNK
NPH = 5
CPP = NCHUNK // NPH
ACC_N = 10240
RPT = ACC_N // NS


def _sc_agg_body(src_hbm, dst_hbm, x_hbm, zeros_hbm, out_hbm,
                 src_v, dst_v, rows_a, rows_b, rows_c,
                 acc_sh, sem_a, sem_b, sem_c):
    c = lax.axis_index("c")
    s = lax.axis_index("s")
    wid = c * NS + s

    pltpu.sync_copy(zeros_hbm.at[s], acc_sh.at[pl.ds(s * RPT, RPT)])
    plsc.subcore_barrier()

    def gather(j, rows, sem):
        return pltpu.async_copy(x_hbm.at[src_v.at[j]], rows, sem)

    def scatter(j, rows):
        pltpu.sync_copy(rows, acc_sh.at[dst_v.at[j]], add=True)

    for p in range(NPH):
        pltpu.sync_copy(src_hbm.at[wid, p], src_v)
        pltpu.sync_copy(dst_hbm.at[wid, p], dst_v)

        gather(0, rows_a, sem_a).wait()

        def body(jj, carry):
            j = 3 * jj
            d_b = gather(j + 1, rows_b, sem_b)
            d_c = gather(j + 2, rows_c, sem_c)
            scatter(j, rows_a)
            d_b.wait()
            d_a = gather(j + 3, rows_a, sem_a)
            scatter(j + 1, rows_b)
            d_c.wait()
            scatter(j + 2, rows_c)
            d_a.wait()
            return carry

        lax.fori_loop(0, (CPP - 1) // 3, body, 0)
        scatter(CPP - 1, rows_a)

    plsc.subcore_barrier()

    pltpu.sync_copy(acc_sh.at[pl.ds(s * RPT, RPT)], out_hbm.at[c, s])


@jax.jit
def _sc_agg(src3d, dst3d, x, zeros):
    mesh = plsc.VectorSubcoreMesh(core_axis_name="c", subcore_axis_name="s",
                                  num_cores=NC, num_subcores=NS)
    f = pl.kernel(
        _sc_agg_body,
        out_type=jax.ShapeDtypeStruct((NC, NS, RPT, D), jnp.float32),
        mesh=mesh,
        scratch_types=[
            pltpu.VMEM((CPP, CHUNK), jnp.int32),
            pltpu.VMEM((CPP, CHUNK), jnp.int32),
            pltpu.VMEM((CHUNK, D), jnp.float32),
            pltpu.VMEM((CHUNK, D), jnp.float32),
            pltpu.VMEM((CHUNK, D), jnp.float32),
            pltpu.VMEM_SHARED((ACC_N, D), jnp.float32),
            pltpu.SemaphoreType.DMA,
            pltpu.SemaphoreType.DMA,
            pltpu.SemaphoreType.DMA,
        ],
    )
    return f(src3d, dst3d, x, zeros)


def _tc_mlp_body(x_ref, p_ref, W1_ref, b1_ref, W2_ref, b2_ref, W3_ref, b3_ref,
                 gamma_ref, beta_ref, o_ref):
    h = x_ref[...] + p_ref[0, :N] + p_ref[1, :N]
    h = jax.nn.sigmoid(
        jnp.dot(h, W1_ref[...], preferred_element_type=jnp.float32)
        + b1_ref[...])
    h = jax.nn.sigmoid(
        jnp.dot(h, W2_ref[...], preferred_element_type=jnp.float32)
        + b2_ref[...])
    h = (jnp.dot(h, W3_ref[...], preferred_element_type=jnp.float32)
         + b3_ref[...])
    h = jnp.where(h >= 0, h, 0.01 * h)
    mean = jnp.mean(h, axis=0, keepdims=True)
    var = jnp.mean(h * h, axis=0, keepdims=True) - mean * mean
    o_ref[...] = ((h - mean) * jax.lax.rsqrt(var + 1e-5) * gamma_ref[...]
                  + beta_ref[...])


@jax.jit
def _tc_mlp(x, partials, W1, b1, W2, b2, W3, b3, gamma, beta):
    return pl.pallas_call(
        _tc_mlp_body,
        out_shape=jax.ShapeDtypeStruct((N, D), jnp.float32),
    )(x, partials, W1, b1.reshape(1, -1), W2, b2.reshape(1, -1),
      W3, b3.reshape(1, -1), gamma.reshape(1, -1), beta.reshape(1, -1))


@jax.jit
def kernel(x, edge_index, W1, b1, W2, b2, W3, b3, gamma, beta):
    src3d = edge_index[0].reshape(NW, NPH, CPP, CHUNK)
    dst3d = edge_index[1].reshape(NW, NPH, CPP, CHUNK)
    zeros = jnp.zeros((NS, RPT, D), jnp.float32)
    out4d = _sc_agg(src3d, dst3d, x, zeros)
    partials = out4d.reshape(NC, ACC_N, D)
    h = _tc_mlp(x, partials, W1, b1, W2, b2, W3, b3, gamma, beta)
    return (h, edge_index)


# speedup vs baseline: 2.6357x; 1.0250x over previous
import jax
import jax.numpy as jnp
from jax import lax
from jax.experimental import pallas as pl
from jax.experimental.pallas import tpu as pltpu
from jax.experimental.pallas import tpu_sc as plsc

N = 10000
E = 320000
D = 128

NC = 2
NS = 16
NW = NC * NS
EPW = E // NW
CHUNK = 80
NCHUNK = EPW // CHUNK
NPH = 5
CPP = NCHUNK // NPH
ACC_N = 10240
RPT = ACC_N // NS


def _sc_agg_body(src_hbm, dst_hbm, x_hbm, zeros_hbm, out_hbm,
                 src_v, dst_v, rows_a, rows_b, rows_c, rows_d,
                 acc_sh, sem_a, sem_b, sem_c, sem_d):
    c = lax.axis_index("c")
    s = lax.axis_index("s")
    wid = c * NS + s

    pltpu.sync_copy(zeros_hbm.at[s], acc_sh.at[pl.ds(s * RPT, RPT)])
    plsc.subcore_barrier()

    def gather(j, rows, sem):
        return pltpu.async_copy(x_hbm.at[src_v.at[j]], rows, sem)

    def scatter(j, rows):
        pltpu.sync_copy(rows, acc_sh.at[dst_v.at[j]], add=True)

    for p in range(NPH):
        pltpu.sync_copy(src_hbm.at[wid, p], src_v)
        pltpu.sync_copy(dst_hbm.at[wid, p], dst_v)

        gather(0, rows_a, sem_a).wait()

        def body(jj, carry):
            j = 4 * jj
            d_b = gather(j + 1, rows_b, sem_b)
            d_c = gather(j + 2, rows_c, sem_c)
            d_d = gather(j + 3, rows_d, sem_d)
            scatter(j, rows_a)
            d_b.wait()
            d_a = gather(j + 4, rows_a, sem_a)
            scatter(j + 1, rows_b)
            d_c.wait()
            scatter(j + 2, rows_c)
            d_d.wait()
            scatter(j + 3, rows_d)
            d_a.wait()
            return carry

        lax.fori_loop(0, (CPP - 1) // 4, body, 0)
        scatter(CPP - 1, rows_a)

    plsc.subcore_barrier()

    pltpu.sync_copy(acc_sh.at[pl.ds(s * RPT, RPT)], out_hbm.at[c, s])


@jax.jit
def _sc_agg(src3d, dst3d, x, zeros):
    mesh = plsc.VectorSubcoreMesh(core_axis_name="c", subcore_axis_name="s",
                                  num_cores=NC, num_subcores=NS)
    f = pl.kernel(
        _sc_agg_body,
        out_type=jax.ShapeDtypeStruct((NC, NS, RPT, D), jnp.float32),
        mesh=mesh,
        scratch_types=[
            pltpu.VMEM((CPP, CHUNK), jnp.int32),
            pltpu.VMEM((CPP, CHUNK), jnp.int32),
            pltpu.VMEM((CHUNK, D), jnp.float32),
            pltpu.VMEM((CHUNK, D), jnp.float32),
            pltpu.VMEM((CHUNK, D), jnp.float32),
            pltpu.VMEM((CHUNK, D), jnp.float32),
            pltpu.VMEM_SHARED((ACC_N, D), jnp.float32),
            pltpu.SemaphoreType.DMA,
            pltpu.SemaphoreType.DMA,
            pltpu.SemaphoreType.DMA,
            pltpu.SemaphoreType.DMA,
        ],
    )
    return f(src3d, dst3d, x, zeros)


def _tc_mlp_body(x_ref, p_ref, W1_ref, b1_ref, W2_ref, b2_ref, W3_ref, b3_ref,
                 gamma_ref, beta_ref, o_ref):
    h = x_ref[...] + p_ref[0, :N] + p_ref[1, :N]
    h = jax.nn.sigmoid(
        jnp.dot(h, W1_ref[...], preferred_element_type=jnp.float32)
        + b1_ref[...])
    h = jax.nn.sigmoid(
        jnp.dot(h, W2_ref[...], preferred_element_type=jnp.float32)
        + b2_ref[...])
    h = (jnp.dot(h, W3_ref[...], preferred_element_type=jnp.float32)
         + b3_ref[...])
    h = jnp.where(h >= 0, h, 0.01 * h)
    mean = jnp.mean(h, axis=0, keepdims=True)
    var = jnp.mean(h * h, axis=0, keepdims=True) - mean * mean
    o_ref[...] = ((h - mean) * jax.lax.rsqrt(var + 1e-5) * gamma_ref[...]
                  + beta_ref[...])


@jax.jit
def _tc_mlp(x, partials, W1, b1, W2, b2, W3, b3, gamma, beta):
    return pl.pallas_call(
        _tc_mlp_body,
        out_shape=jax.ShapeDtypeStruct((N, D), jnp.float32),
    )(x, partials, W1, b1.reshape(1, -1), W2, b2.reshape(1, -1),
      W3, b3.reshape(1, -1), gamma.reshape(1, -1), beta.reshape(1, -1))


@jax.jit
def kernel(x, edge_index, W1, b1, W2, b2, W3, b3, gamma, beta):
    src3d = edge_index[0].reshape(NW, NPH, CPP, CHUNK)
    dst3d = edge_index[1].reshape(NW, NPH, CPP, CHUNK)
    zeros = jnp.zeros((NS, RPT, D), jnp.float32)
    out4d = _sc_agg(src3d, dst3d, x, zeros)
    partials = out4d.reshape(NC, ACC_N, D)
    h = _tc_mlp(x, partials, W1, b1, W2, b2, W3, b3, gamma, beta)
    return (h, edge_index)


# speedup vs baseline: 2.6481x; 1.0047x over previous
import jax
import jax.numpy as jnp
from jax import lax
from jax.experimental import pallas as pl
from jax.experimental.pallas import tpu as pltpu
from jax.experimental.pallas import tpu_sc as plsc

N = 10000
E = 320000
D = 128

NC = 2
NS = 16
NW = NC * NS
EPW = E // NW
CHUNK = 80
NCHUNK = EPW // CHUNK
NPH = 5
CPP = NCHUNK // NPH
ACC_N = 10240
RPT = ACC_N // NS


def _sc_agg_body(src_hbm, dst_hbm, x_hbm, zeros_hbm, out_hbm,
                 src_v, dst_v, rows_a, rows_b, rows_c, rows_d,
                 acc_sh, sem_a, sem_b, sem_c, sem_d, sem_sa, sem_sb):
    c = lax.axis_index("c")
    s = lax.axis_index("s")
    wid = c * NS + s

    pltpu.sync_copy(zeros_hbm.at[s], acc_sh.at[pl.ds(s * RPT, RPT)])
    plsc.subcore_barrier()

    def gather(j, rows, sem):
        return pltpu.async_copy(x_hbm.at[src_v.at[j]], rows, sem)

    def scatter(j, rows):
        pltpu.sync_copy(rows, acc_sh.at[dst_v.at[j]], add=True)

    def ascatter(j, rows, sem):
        return pltpu.async_copy(rows, acc_sh.at[dst_v.at[j]], sem, add=True)

    for p in range(NPH):
        pltpu.sync_copy(src_hbm.at[wid, p], src_v)
        pltpu.sync_copy(dst_hbm.at[wid, p], dst_v)

        gather(0, rows_a, sem_a).wait()

        def body(jj, carry):
            j = 4 * jj
            d_b = gather(j + 1, rows_b, sem_b)
            d_c = gather(j + 2, rows_c, sem_c)
            d_d = gather(j + 3, rows_d, sem_d)
            s_a = ascatter(j, rows_a, sem_sa)
            d_b.wait()
            s_b = ascatter(j + 1, rows_b, sem_sb)
            s_a.wait()
            d_a = gather(j + 4, rows_a, sem_a)
            d_c.wait()
            s_c = ascatter(j + 2, rows_c, sem_sa)
            s_b.wait()
            d_d.wait()
            s_d = ascatter(j + 3, rows_d, sem_sb)
            s_c.wait()
            s_d.wait()
            d_a.wait()
            return carry

        lax.fori_loop(0, (CPP - 1) // 4, body, 0)
        scatter(CPP - 1, rows_a)

    plsc.subcore_barrier()

    pltpu.sync_copy(acc_sh.at[pl.ds(s * RPT, RPT)], out_hbm.at[c, s])


@jax.jit
def _sc_agg(src3d, dst3d, x, zeros):
    mesh = plsc.VectorSubcoreMesh(core_axis_name="c", subcore_axis_name="s",
                                  num_cores=NC, num_subcores=NS)
    f = pl.kernel(
        _sc_agg_body,
        out_type=jax.ShapeDtypeStruct((NC, NS, RPT, D), jnp.float32),
        mesh=mesh,
        scratch_types=[
            pltpu.VMEM((CPP, CHUNK), jnp.int32),
            pltpu.VMEM((CPP, CHUNK), jnp.int32),
            pltpu.VMEM((CHUNK, D), jnp.float32),
            pltpu.VMEM((CHUNK, D), jnp.float32),
            pltpu.VMEM((CHUNK, D), jnp.float32),
            pltpu.VMEM((CHUNK, D), jnp.float32),
            pltpu.VMEM_SHARED((ACC_N, D), jnp.float32),
            pltpu.SemaphoreType.DMA,
            pltpu.SemaphoreType.DMA,
            pltpu.SemaphoreType.DMA,
            pltpu.SemaphoreType.DMA,
            pltpu.SemaphoreType.DMA,
            pltpu.SemaphoreType.DMA,
        ],
    )
    return f(src3d, dst3d, x, zeros)


def _tc_mlp_body(x_ref, p_ref, W1_ref, b1_ref, W2_ref, b2_ref, W3_ref, b3_ref,
                 gamma_ref, beta_ref, o_ref):
    h = x_ref[...] + p_ref[0, :N] + p_ref[1, :N]
    h = jax.nn.sigmoid(
        jnp.dot(h, W1_ref[...], preferred_element_type=jnp.float32)
        + b1_ref[...])
    h = jax.nn.sigmoid(
        jnp.dot(h, W2_ref[...], preferred_element_type=jnp.float32)
        + b2_ref[...])
    h = (jnp.dot(h, W3_ref[...], preferred_element_type=jnp.float32)
         + b3_ref[...])
    h = jnp.where(h >= 0, h, 0.01 * h)
    mean = jnp.mean(h, axis=0, keepdims=True)
    var = jnp.mean(h * h, axis=0, keepdims=True) - mean * mean
    o_ref[...] = ((h - mean) * jax.lax.rsqrt(var + 1e-5) * gamma_ref[...]
                  + beta_ref[...])


@jax.jit
def _tc_mlp(x, partials, W1, b1, W2, b2, W3, b3, gamma, beta):
    return pl.pallas_call(
        _tc_mlp_body,
        out_shape=jax.ShapeDtypeStruct((N, D), jnp.float32),
    )(x, partials, W1, b1.reshape(1, -1), W2, b2.reshape(1, -1),
      W3, b3.reshape(1, -1), gamma.reshape(1, -1), beta.reshape(1, -1))


@jax.jit
def kernel(x, edge_index, W1, b1, W2, b2, W3, b3, gamma, beta):
    src3d = edge_index[0].reshape(NW, NPH, CPP, CHUNK)
    dst3d = edge_index[1].reshape(NW, NPH, CPP, CHUNK)
    zeros = jnp.zeros((NS, RPT, D), jnp.float32)
    out4d = _sc_agg(src3d, dst3d, x, zeros)
    partials = out4d.reshape(NC, ACC_N, D)
    h = _tc_mlp(x, partials, W1, b1, W2, b2, W3, b3, gamma, beta)
    return (h, edge_index)
